# baseline (speedup 1.0000x reference)
import functools

import jax
import jax.numpy as jnp
from jax import lax
from jax.experimental import pallas as pl
from jax.experimental.pallas import tpu as pltpu
from jax.experimental.pallas import tpu_sc as plsc

_B = 16384
_F = 26
_V = 100000
_D = 64
_NUM = 13
_EPS = 1e-5
_NPAD = 128

_H0, _H1, _H2 = 1024, 512, 256

_NC = 2
_NS = 16
_NW = _NC * _NS
_ROWS = _B * _F
_RPW = _ROWS // _NW
_CHUNK = 128
_NCH = _RPW // _CHUNK
_NB = 4

_BT = 1024
_GRID = _B // _BT


def _sc_gather(tab2d, idx2d):
  mesh = plsc.VectorSubcoreMesh(core_axis_name="c", subcore_axis_name="s")

  @functools.partial(
      pl.kernel,
      mesh=mesh,
      compiler_params=pltpu.CompilerParams(use_tc_tiling_on_sc=False),
      out_type=jax.ShapeDtypeStruct((_ROWS, _D), jnp.float32),
      scratch_types=[
          pltpu.VMEM((_NCH, _CHUNK), jnp.int32),
          pltpu.VMEM((_NB, _CHUNK, _D), jnp.float32),
          pltpu.SemaphoreType.DMA,
          pltpu.SemaphoreType.DMA,
      ],
  )
  def k(tab_hbm, idx_hbm, out_hbm, idx_v, rows_v, gsem, osem):
    w = lax.axis_index("s") * _NC + lax.axis_index("c")
    pltpu.sync_copy(idx_hbm.at[pl.ds(w * _NCH, _NCH)], idx_v)
    base = w * _RPW

    def body(jj, carry):
      @pl.when(jj > 0)
      def _drain_prev():
        for kb in range(_NB):
          pltpu.make_async_copy(
              rows_v.at[kb], out_hbm.at[pl.ds(0, _CHUNK)], osem).wait()
      handles = []
      for kb in range(_NB):
        j = jj * _NB + kb
        handles.append(
            pltpu.async_copy(tab_hbm.at[idx_v.at[j]], rows_v.at[kb], gsem))
      for h in handles:
        h.wait()
      for kb in range(_NB):
        j = jj * _NB + kb
        pltpu.async_copy(
            rows_v.at[kb], out_hbm.at[pl.ds(base + j * _CHUNK, _CHUNK)], osem)
      return carry

    lax.fori_loop(0, _NCH // _NB, body, 0)
    for kb in range(_NB):
      pltpu.make_async_copy(
          rows_v.at[kb], out_hbm.at[pl.ds(0, _CHUNK)], osem).wait()

  return k(tab2d, idx2d)


def _colstats(a):
  return jnp.concatenate(
      [jnp.sum(a, axis=0, keepdims=True),
       jnp.sum(a * a, axis=0, keepdims=True)], axis=0)


def _bn_scale_shift(sums_ref, g_ref, be_ref):
  m = sums_ref[0, :] * (1.0 / _B)
  mq = sums_ref[1, :] * (1.0 / _B)
  var = jnp.maximum(mq - m * m, 0.0)
  sc = g_ref[0, :] * lax.rsqrt(var + _EPS)
  sh = be_ref[0, :] - m * sc
  return sc, sh


def _accum(i, ref, part):
  @pl.when(i == 0)
  def _():
    ref[...] = jnp.zeros_like(ref)
  ref[...] += part


def _stats_body(x_ref, o_ref):
  _accum(pl.program_id(0), o_ref, _colstats(x_ref[...]))


def _stats0(xn_pad):
  return pl.pallas_call(
      _stats_body,
      grid=(_GRID,),
      in_specs=[pl.BlockSpec((_BT, _NPAD), lambda i: (i, 0))],
      out_specs=pl.BlockSpec((2, _NPAD), lambda i: (0, 0)),
      out_shape=jax.ShapeDtypeStruct((2, _NPAD), jnp.float32),
  )(xn_pad)


def _l0_body(emb_ref, xn_ref, s0_ref, gn_ref, bn_ref, weT_ref, wnT_ref,
             b_ref, a_ref, s1_ref):
  sc, sh = _bn_scale_shift(s0_ref, gn_ref, bn_ref)
  y_n = xn_ref[...] * sc[None, :] + sh[None, :]
  z = (jnp.dot(emb_ref[...], weT_ref[...], preferred_element_type=jnp.float32)
       + jnp.dot(y_n, wnT_ref[...], preferred_element_type=jnp.float32)
       + b_ref[...])
  a = jnp.maximum(z, 0.0)
  a_ref[...] = a
  _accum(pl.program_id(0), s1_ref, _colstats(a))


def _layer0(emb, xn_pad, s0, gn, bn, weT, wnT, br):
  kd = _F * _D
  return pl.pallas_call(
      _l0_body,
      grid=(_GRID,),
      in_specs=[
          pl.BlockSpec((_BT, kd), lambda i: (i, 0)),
          pl.BlockSpec((_BT, _NPAD), lambda i: (i, 0)),
          pl.BlockSpec((2, _NPAD), lambda i: (0, 0)),
          pl.BlockSpec((1, _NPAD), lambda i: (0, 0)),
          pl.BlockSpec((1, _NPAD), lambda i: (0, 0)),
          pl.BlockSpec((kd, _H0), lambda i: (0, 0)),
          pl.BlockSpec((_NPAD, _H0), lambda i: (0, 0)),
          pl.BlockSpec((1, _H0), lambda i: (0, 0)),
      ],
      out_specs=[
          pl.BlockSpec((_BT, _H0), lambda i: (i, 0)),
          pl.BlockSpec((2, _H0), lambda i: (0, 0)),
      ],
      out_shape=[
          jax.ShapeDtypeStruct((_B, _H0), jnp.float32),
          jax.ShapeDtypeStruct((2, _H0), jnp.float32),
      ],
  )(emb, xn_pad, s0, gn, bn, weT, wnT, br)


def _mid_body(a_ref, sin_ref, g_ref, be_ref, wT_ref, b_ref, ao_ref, sout_ref):
  sc, sh = _bn_scale_shift(sin_ref, g_ref, be_ref)
  y = a_ref[...] * sc[None, :] + sh[None, :]
  z = jnp.dot(y, wT_ref[...], preferred_element_type=jnp.float32) + b_ref[...]
  a = jnp.maximum(z, 0.0)
  ao_ref[...] = a
  _accum(pl.program_id(0), sout_ref, _colstats(a))


def _mid(a, sin, g, be, wT, br, h_in, h_out):
  return pl.pallas_call(
      _mid_body,
      grid=(_GRID,),
      in_specs=[
          pl.BlockSpec((_BT, h_in), lambda i: (i, 0)),
          pl.BlockSpec((2, h_in), lambda i: (0, 0)),
          pl.BlockSpec((1, h_in), lambda i: (0, 0)),
          pl.BlockSpec((1, h_in), lambda i: (0, 0)),
          pl.BlockSpec((h_in, h_out), lambda i: (0, 0)),
          pl.BlockSpec((1, h_out), lambda i: (0, 0)),
      ],
      out_specs=[
          pl.BlockSpec((_BT, h_out), lambda i: (i, 0)),
          pl.BlockSpec((2, h_out), lambda i: (0, 0)),
      ],
      out_shape=[
          jax.ShapeDtypeStruct((_B, h_out), jnp.float32),
          jax.ShapeDtypeStruct((2, h_out), jnp.float32),
      ],
  )(a, sin, g, be, wT, br)


def _fin_body(a_ref, sin_ref, g_ref, be_ref, wd_ref, bd_ref, o_ref):
  sc, sh = _bn_scale_shift(sin_ref, g_ref, be_ref)
  y = a_ref[...] * sc[None, :] + sh[None, :]
  d = jnp.dot(y, wd_ref[...], preferred_element_type=jnp.float32) + bd_ref[...]
  o_ref[...] = 1.0 / (1.0 + jnp.exp(-d))


def _fin(a, sin, g, be, wd, bd):
  return pl.pallas_call(
      _fin_body,
      grid=(_GRID,),
      in_specs=[
          pl.BlockSpec((_BT, _H2), lambda i: (i, 0)),
          pl.BlockSpec((2, _H2), lambda i: (0, 0)),
          pl.BlockSpec((1, _H2), lambda i: (0, 0)),
          pl.BlockSpec((1, _H2), lambda i: (0, 0)),
          pl.BlockSpec((_H2, 1), lambda i: (0, 0)),
          pl.BlockSpec((1, 1), lambda i: (0, 0)),
      ],
      out_specs=pl.BlockSpec((_BT, 1), lambda i: (i, 0)),
      out_shape=jax.ShapeDtypeStruct((_B, 1), jnp.float32),
  )(a, sin, g, be, wd, bd)


def kernel(x_categorical, x_numerical, tables, bn_num_gamma, bn_num_beta,
           W0, b0, g0, be0, W1, b1, g1, be1, W2, b2, g2, be2, Wo, bo):
  tab2d = tables.reshape(_F * _V, _D)
  offs = (jnp.arange(_F, dtype=jnp.int32) * _V)[None, :]
  idx2d = (x_categorical.astype(jnp.int32) + offs).reshape(_NW * _NCH, _CHUNK)
  emb = _sc_gather(tab2d, idx2d).reshape(_B, _F * _D)

  xn_pad = jnp.pad(x_numerical, ((0, 0), (0, _NPAD - _NUM)))
  gn = jnp.pad(bn_num_gamma, (0, _NPAD - _NUM)).reshape(1, _NPAD)
  bn = jnp.pad(bn_num_beta, (0, _NPAD - _NUM)).reshape(1, _NPAD)
  s0 = _stats0(xn_pad)

  weT = W0[:, :_F * _D].T
  wnT = jnp.pad(W0[:, _F * _D:], ((0, 0), (0, _NPAD - _NUM))).T
  a1, s1 = _layer0(emb, xn_pad, s0, gn, bn, weT, wnT, b0.reshape(1, _H0))
  a2, s2 = _mid(a1, s1, g0.reshape(1, _H0), be0.reshape(1, _H0),
                W1.T, b1.reshape(1, _H1), _H0, _H1)
  a3, s3 = _mid(a2, s2, g1.reshape(1, _H1), be1.reshape(1, _H1),
                W2.T, b2.reshape(1, _H2), _H1, _H2)
  wd = (Wo[1] - Wo[0]).reshape(_H2, 1)
  bd = (bo[1] - bo[0]).reshape(1, 1)
  out = _fin(a3, s3, g2.reshape(1, _H2), be2.reshape(1, _H2), wd, bd)
  return out.reshape(_B)


# speedup vs baseline: 2.4593x; 2.4593x over previous
import functools

import jax
import jax.numpy as jnp
from jax import lax
from jax.experimental import pallas as pl
from jax.experimental.pallas import tpu as pltpu
from jax.experimental.pallas import tpu_sc as plsc

_B = 16384
_F = 26
_V = 100000
_D = 64
_NUM = 13
_EPS = 1e-5
_NPAD = 128

_H0, _H1, _H2 = 1024, 512, 256

_NC = 2
_NS = 16
_NW = _NC * _NS
_BPW = _B // _NW
_CS = 16
_NCHK = _BPW // _CS
_FP = 32

_BT = 1024
_GRID = _B // _BT


def _sc_gather(tab2d, idxT):
  mesh = plsc.VectorSubcoreMesh(core_axis_name="c", subcore_axis_name="s")

  @functools.partial(
      pl.kernel,
      mesh=mesh,
      out_type=jax.ShapeDtypeStruct((_B, _F * _D), jnp.float32),
      scratch_types=[
          pltpu.VMEM((_FP, _BPW), jnp.int32),
          pltpu.VMEM((2, _CS, _F * _D), jnp.float32),
          pltpu.SemaphoreType.DMA,
          pltpu.SemaphoreType.DMA,
          pltpu.SemaphoreType.DMA,
      ],
  )
  def k(tab_hbm, idx_hbm, out_hbm, idx_v, buf, gsem, os0, os1):
    w = lax.axis_index("s") * _NC + lax.axis_index("c")
    bw = w * _BPW
    pltpu.sync_copy(idx_hbm.at[:, pl.ds(bw, _BPW)], idx_v)
    osems = (os0, os1)

    def chunk(c, carry):
      c16 = c * _CS
      par = c % 2
      is0 = par == 0
      for b in range(2):
        @pl.when((c > 1) & (par == b))
        def _():
          pltpu.make_async_copy(
              buf.at[b], out_hbm.at[pl.ds(0, _CS), :], osems[b]).wait()

      def field(f, carry2):
        idx16 = idx_v[f, pl.ds(c16, _CS)]
        for i in range(_CS):
          pltpu.async_copy(
              tab_hbm.at[idx16[i]],
              buf.at[par, i, pl.ds(f * _D, _D)],
              gsem)
        return carry2

      lax.fori_loop(0, _F, field, 0)
      pltpu.make_async_copy(
          out_hbm.at[pl.ds(0, _CS), :], buf.at[par], gsem).wait()
      for b in range(2):
        @pl.when(par == b)
        def _():
          pltpu.async_copy(
              buf.at[b], out_hbm.at[pl.ds(bw + c16, _CS), :], osems[b])
      return carry

    lax.fori_loop(0, _NCHK, chunk, 0)
    for b in range(2):
      pltpu.make_async_copy(
          buf.at[b], out_hbm.at[pl.ds(0, _CS), :], osems[b]).wait()

  return k(tab2d, idxT)


def _colstats(a):
  return jnp.concatenate(
      [jnp.sum(a, axis=0, keepdims=True),
       jnp.sum(a * a, axis=0, keepdims=True)], axis=0)


def _bn_scale_shift(sums_ref, g_ref, be_ref):
  m = sums_ref[0, :] * (1.0 / _B)
  mq = sums_ref[1, :] * (1.0 / _B)
  var = jnp.maximum(mq - m * m, 0.0)
  sc = g_ref[0, :] * lax.rsqrt(var + _EPS)
  sh = be_ref[0, :] - m * sc
  return sc, sh


def _accum(i, ref, part):
  @pl.when(i == 0)
  def _():
    ref[...] = jnp.zeros_like(ref)
  ref[...] += part


def _stats_body(x_ref, o_ref):
  _accum(pl.program_id(0), o_ref, _colstats(x_ref[...]))


def _stats0(xn_pad):
  return pl.pallas_call(
      _stats_body,
      grid=(_GRID,),
      in_specs=[pl.BlockSpec((_BT, _NPAD), lambda i: (i, 0))],
      out_specs=pl.BlockSpec((2, _NPAD), lambda i: (0, 0)),
      out_shape=jax.ShapeDtypeStruct((2, _NPAD), jnp.float32),
  )(xn_pad)


def _l0_body(emb_ref, xn_ref, s0_ref, gn_ref, bn_ref, weT_ref, wnT_ref,
             b_ref, a_ref, s1_ref):
  sc, sh = _bn_scale_shift(s0_ref, gn_ref, bn_ref)
  y_n = xn_ref[...] * sc[None, :] + sh[None, :]
  z = (jnp.dot(emb_ref[...], weT_ref[...], preferred_element_type=jnp.float32)
       + jnp.dot(y_n, wnT_ref[...], preferred_element_type=jnp.float32)
       + b_ref[...])
  a = jnp.maximum(z, 0.0)
  a_ref[...] = a
  _accum(pl.program_id(0), s1_ref, _colstats(a))


def _layer0(emb, xn_pad, s0, gn, bn, weT, wnT, br):
  kd = _F * _D
  return pl.pallas_call(
      _l0_body,
      grid=(_GRID,),
      in_specs=[
          pl.BlockSpec((_BT, kd), lambda i: (i, 0)),
          pl.BlockSpec((_BT, _NPAD), lambda i: (i, 0)),
          pl.BlockSpec((2, _NPAD), lambda i: (0, 0)),
          pl.BlockSpec((1, _NPAD), lambda i: (0, 0)),
          pl.BlockSpec((1, _NPAD), lambda i: (0, 0)),
          pl.BlockSpec((kd, _H0), lambda i: (0, 0)),
          pl.BlockSpec((_NPAD, _H0), lambda i: (0, 0)),
          pl.BlockSpec((1, _H0), lambda i: (0, 0)),
      ],
      out_specs=[
          pl.BlockSpec((_BT, _H0), lambda i: (i, 0)),
          pl.BlockSpec((2, _H0), lambda i: (0, 0)),
      ],
      out_shape=[
          jax.ShapeDtypeStruct((_B, _H0), jnp.float32),
          jax.ShapeDtypeStruct((2, _H0), jnp.float32),
      ],
  )(emb, xn_pad, s0, gn, bn, weT, wnT, br)


def _mid_body(a_ref, sin_ref, g_ref, be_ref, wT_ref, b_ref, ao_ref, sout_ref):
  sc, sh = _bn_scale_shift(sin_ref, g_ref, be_ref)
  y = a_ref[...] * sc[None, :] + sh[None, :]
  z = jnp.dot(y, wT_ref[...], preferred_element_type=jnp.float32) + b_ref[...]
  a = jnp.maximum(z, 0.0)
  ao_ref[...] = a
  _accum(pl.program_id(0), sout_ref, _colstats(a))


def _mid(a, sin, g, be, wT, br, h_in, h_out):
  return pl.pallas_call(
      _mid_body,
      grid=(_GRID,),
      in_specs=[
          pl.BlockSpec((_BT, h_in), lambda i: (i, 0)),
          pl.BlockSpec((2, h_in), lambda i: (0, 0)),
          pl.BlockSpec((1, h_in), lambda i: (0, 0)),
          pl.BlockSpec((1, h_in), lambda i: (0, 0)),
          pl.BlockSpec((h_in, h_out), lambda i: (0, 0)),
          pl.BlockSpec((1, h_out), lambda i: (0, 0)),
      ],
      out_specs=[
          pl.BlockSpec((_BT, h_out), lambda i: (i, 0)),
          pl.BlockSpec((2, h_out), lambda i: (0, 0)),
      ],
      out_shape=[
          jax.ShapeDtypeStruct((_B, h_out), jnp.float32),
          jax.ShapeDtypeStruct((2, h_out), jnp.float32),
      ],
  )(a, sin, g, be, wT, br)


def _fin_body(a_ref, sin_ref, g_ref, be_ref, wd_ref, bd_ref, o_ref):
  sc, sh = _bn_scale_shift(sin_ref, g_ref, be_ref)
  y = a_ref[...] * sc[None, :] + sh[None, :]
  d = jnp.dot(y, wd_ref[...], preferred_element_type=jnp.float32) + bd_ref[...]
  o_ref[...] = 1.0 / (1.0 + jnp.exp(-d))


def _fin(a, sin, g, be, wd, bd):
  return pl.pallas_call(
      _fin_body,
      grid=(_GRID,),
      in_specs=[
          pl.BlockSpec((_BT, _H2), lambda i: (i, 0)),
          pl.BlockSpec((2, _H2), lambda i: (0, 0)),
          pl.BlockSpec((1, _H2), lambda i: (0, 0)),
          pl.BlockSpec((1, _H2), lambda i: (0, 0)),
          pl.BlockSpec((_H2, 1), lambda i: (0, 0)),
          pl.BlockSpec((1, 1), lambda i: (0, 0)),
      ],
      out_specs=pl.BlockSpec((_BT, 1), lambda i: (i, 0)),
      out_shape=jax.ShapeDtypeStruct((_B, 1), jnp.float32),
  )(a, sin, g, be, wd, bd)


def kernel(x_categorical, x_numerical, tables, bn_num_gamma, bn_num_beta,
           W0, b0, g0, be0, W1, b1, g1, be1, W2, b2, g2, be2, Wo, bo):
  tab2d = tables.reshape(_F * _V, _D)
  offs = (jnp.arange(_F, dtype=jnp.int32) * _V)[None, :]
  flat = x_categorical.astype(jnp.int32) + offs
  idxT = jnp.pad(flat.T, ((0, _FP - _F), (0, 0)))
  emb = _sc_gather(tab2d, idxT)

  xn_pad = jnp.pad(x_numerical, ((0, 0), (0, _NPAD - _NUM)))
  gn = jnp.pad(bn_num_gamma, (0, _NPAD - _NUM)).reshape(1, _NPAD)
  bn = jnp.pad(bn_num_beta, (0, _NPAD - _NUM)).reshape(1, _NPAD)
  s0 = _stats0(xn_pad)

  weT = W0[:, :_F * _D].T
  wnT = jnp.pad(W0[:, _F * _D:], ((0, 0), (0, _NPAD - _NUM))).T
  a1, s1 = _layer0(emb, xn_pad, s0, gn, bn, weT, wnT, b0.reshape(1, _H0))
  a2, s2 = _mid(a1, s1, g0.reshape(1, _H0), be0.reshape(1, _H0),
                W1.T, b1.reshape(1, _H1), _H0, _H1)
  a3, s3 = _mid(a2, s2, g1.reshape(1, _H1), be1.reshape(1, _H1),
                W2.T, b2.reshape(1, _H2), _H1, _H2)
  wd = (Wo[1] - Wo[0]).reshape(_H2, 1)
  bd = (bo[1] - bo[0]).reshape(1, 1)
  out = _fin(a3, s3, g2.reshape(1, _H2), be2.reshape(1, _H2), wd, bd)
  return out.reshape(_B)
